# baseline (speedup 1.0000x reference)
import functools

import jax
import jax.numpy as jnp
from jax import lax
from jax.experimental import pallas as pl
from jax.experimental.pallas import tpu as pltpu
from jax.experimental.pallas import tpu_sc as plsc

N = 10000
E = 320000
NC = 2
NS = 16
NW = NC * NS
EPW = E // NW
CH = 112
NCHUNK = 90
EPT = NCHUNK * CH
NP = N + 16
RPT = 624
REMZ = NP - NS * RPT
REMO = N - NS * RPT

BLK = 1000
GRID = N // BLK

@functools.cache
def _mesh():
    return plsc.VectorSubcoreMesh(core_axis_name="c", subcore_axis_name="s",
                                  num_cores=NC, num_subcores=NS)


@functools.cache
def _make_agg(Dp):
    @functools.partial(
        pl.kernel,
        mesh=_mesh(),
        compiler_params=pltpu.CompilerParams(use_tc_tiling_on_sc=False),
        out_type=jax.ShapeDtypeStruct((NC, N, Dp), jnp.float32),
        scratch_types=[
            pltpu.VMEM((NCHUNK, CH), jnp.int32),
            pltpu.VMEM((NCHUNK, CH), jnp.int32),
            pltpu.VMEM((CH, Dp), jnp.float32),
            pltpu.VMEM((CH, Dp), jnp.float32),
            pltpu.VMEM_SHARED((NP, Dp), jnp.float32),
            pltpu.SemaphoreType.DMA,
            pltpu.SemaphoreType.DMA,
        ],
    )
    def agg(table_hbm, src_hbm, dst_hbm, zeros_hbm, out_hbm,
            src_v, dst_v, b0, b1, acc, g0, g1):
        bufs = [b0, b1]
        gsem = [g0, g1]
        c = lax.axis_index("c")
        s = lax.axis_index("s")
        wid = s * NC + c
        pltpu.sync_copy(zeros_hbm, acc.at[pl.ds(s * RPT, RPT)])

        @pl.when(s == 0)
        def _():
            pltpu.sync_copy(zeros_hbm.at[pl.ds(0, REMZ)],
                            acc.at[pl.ds(NS * RPT, REMZ)])

        pltpu.sync_copy(src_hbm.at[wid], src_v)
        pltpu.sync_copy(dst_hbm.at[wid], dst_v)
        plsc.subcore_barrier()

        def issue_g(i, k):
            pltpu.async_copy(table_hbm.at[src_v.at[i]], bufs[k], gsem[k])

        def wait_g(k):
            pltpu.make_async_copy(table_hbm.at[src_v.at[0]], bufs[k],
                                  gsem[k]).wait()

        issue_g(0, 0)

        def pair(j, _):
            i0 = 2 * j
            issue_g(i0 + 1, 1)
            wait_g(0)
            pltpu.sync_copy(bufs[0], acc.at[dst_v.at[i0]], add=True)

            @pl.when(i0 + 2 < NCHUNK)
            def _():
                issue_g(i0 + 2, 0)

            wait_g(1)
            pltpu.sync_copy(bufs[1], acc.at[dst_v.at[i0 + 1]], add=True)
            return ()

        lax.fori_loop(0, NCHUNK // 2, pair, (), unroll=False)
        if NCHUNK % 2:
            wait_g(0)
            pltpu.sync_copy(bufs[0], acc.at[dst_v.at[NCHUNK - 1]], add=True)
        plsc.subcore_barrier()
        pltpu.sync_copy(acc.at[pl.ds(s * RPT, RPT)],
                        out_hbm.at[c, pl.ds(s * RPT, RPT)])

        @pl.when(s == 0)
        def _():
            pltpu.sync_copy(acc.at[pl.ds(NS * RPT, REMO)],
                            out_hbm.at[c, pl.ds(NS * RPT, REMO)])

    return agg


@functools.cache
def _make_deg():
    @functools.partial(
        pl.kernel,
        mesh=_mesh(),
        compiler_params=pltpu.CompilerParams(use_tc_tiling_on_sc=False),
        out_type=jax.ShapeDtypeStruct((NC, N, 16), jnp.float32),
        scratch_types=[
            pltpu.VMEM((NCHUNK, CH), jnp.int32),
            pltpu.VMEM((CH, 16), jnp.float32),
            pltpu.VMEM_SHARED((NP, 16), jnp.float32),
            pltpu.SemaphoreType.DMA,
        ],
    )
    def deg(dst_hbm, ones_hbm, zeros_hbm, out_hbm, dst_v, ones_v, acc, sem):
        c = lax.axis_index("c")
        s = lax.axis_index("s")
        wid = s * NC + c
        pltpu.sync_copy(zeros_hbm, acc.at[pl.ds(s * RPT, RPT)])

        @pl.when(s == 0)
        def _():
            pltpu.sync_copy(zeros_hbm.at[pl.ds(0, REMZ)],
                            acc.at[pl.ds(NS * RPT, REMZ)])

        pltpu.sync_copy(ones_hbm, ones_v)
        pltpu.sync_copy(dst_hbm.at[wid], dst_v)
        plsc.subcore_barrier()

        def fire(i, _):
            pltpu.async_copy(ones_v, acc.at[dst_v.at[i]], sem, add=True)
            return ()

        lax.fori_loop(0, NCHUNK, fire, (), unroll=False)

        def drain(i, _):
            pltpu.make_async_copy(ones_v, acc.at[dst_v.at[0]], sem).wait()
            return ()

        lax.fori_loop(0, NCHUNK, drain, (), unroll=False)
        plsc.subcore_barrier()
        pltpu.sync_copy(acc.at[pl.ds(s * RPT, RPT)],
                        out_hbm.at[c, pl.ds(s * RPT, RPT)])

        @pl.when(s == 0)
        def _():
            pltpu.sync_copy(acc.at[pl.ds(NS * RPT, REMO)],
                            out_hbm.at[c, pl.ds(NS * RPT, REMO)])

    return deg


def _dinv_of(degp_ref):
    deg = degp_ref[0, :, 0:1] + degp_ref[1, :, 0:1] + 1.0
    return lax.rsqrt(deg)


def _t1_body(x_ref, w0_ref, degp_ref, g1_ref):
    h = jnp.dot(x_ref[...], w0_ref[...], preferred_element_type=jnp.float32)
    g1_ref[...] = _dinv_of(degp_ref) * h


def _t2_body(g1_ref, a1_ref, degp_ref, w1_ref, b0_ref, g2_ref):
    dinv = _dinv_of(degp_ref)
    out1 = jnp.maximum(
        dinv * (a1_ref[0] + a1_ref[1] + g1_ref[...]) + b0_ref[...], 0.0)
    g2_ref[...] = dinv * jnp.dot(out1, w1_ref[...],
                                 preferred_element_type=jnp.float32)


def _t3_body(g2_ref, a2_ref, degp_ref, b1_ref, g3_ref):
    dinv = _dinv_of(degp_ref)
    logits = dinv * (a2_ref[0] + a2_ref[1] + g2_ref[...]) + b1_ref[...]
    pad = lax.broadcasted_iota(jnp.int32, logits.shape, 1) >= 40
    logits = jnp.where(pad, -1e30, logits)
    m = jnp.max(logits, axis=1, keepdims=True)
    e = jnp.exp(logits - m)
    p = e / jnp.sum(e, axis=1, keepdims=True)
    g3_ref[...] = dinv * p


def _t4_body(g3_ref, a3_ref, degp_ref, wc0_ref, bc0_ref, wc1_ref, g4_ref):
    dinv = _dinv_of(degp_ref)
    z = dinv * (a3_ref[0] + a3_ref[1] + g3_ref[...])
    h2 = jnp.maximum(
        jnp.dot(z, wc0_ref[...], preferred_element_type=jnp.float32)
        + bc0_ref[...], 0.0)
    g4_ref[...] = dinv * jnp.dot(h2, wc1_ref[...],
                                 preferred_element_type=jnp.float32)


def _t5_body(g4_ref, a4_ref, degp_ref, bc1_ref, out_ref):
    dinv = _dinv_of(degp_ref)
    out_ref[...] = (dinv * (a4_ref[0] + a4_ref[1] + g4_ref[...]))[:, :40] \
        + bc1_ref[...]


def _row_spec(d):
    return pl.BlockSpec((BLK, d), lambda i: (i, 0))


def _agg_spec(d):
    return pl.BlockSpec((2, BLK, d), lambda i: (0, i, 0))


def _full_spec(shape):
    nd = len(shape)
    return pl.BlockSpec(shape, lambda i: (0,) * nd)


_DEGP = _agg_spec(16)


def _tc_call(body, in_specs, out_d):
    return pl.pallas_call(
        body,
        grid=(GRID,),
        in_specs=in_specs,
        out_specs=_row_spec(out_d),
        out_shape=jax.ShapeDtypeStruct((N, out_d), jnp.float32),
    )


_t1 = _tc_call(_t1_body, [_row_spec(128), _full_spec((128, 128)), _DEGP], 128)
_t2 = _tc_call(_t2_body, [_row_spec(128), _agg_spec(128), _DEGP,
                          _full_spec((128, 48)), _full_spec((1, 128))], 48)
_t3 = _tc_call(_t3_body, [_row_spec(48), _agg_spec(48), _DEGP,
                          _full_spec((1, 48))], 48)
_t4 = _tc_call(_t4_body, [_row_spec(48), _agg_spec(48), _DEGP,
                          _full_spec((48, 128)), _full_spec((1, 128)),
                          _full_spec((128, 48))], 48)
_t5 = _tc_call(_t5_body, [_row_spec(48), _agg_spec(48), _DEGP,
                          _full_spec((1, 40))], 40)


def kernel(x, edge_index, W0, b0, W1, b1, Wc0, bc0, Wc1, bc1):
    def edge_split(nparts, nchunk):
        npad = nchunk * CH - E // nparts
        s2 = edge_index[0].reshape(nparts, E // nparts)
        d2 = edge_index[1].reshape(nparts, E // nparts)
        sp = jnp.pad(s2, ((0, 0), (0, npad))).reshape(nparts, nchunk, CH)
        dummy = (N + jnp.arange(nparts, dtype=jnp.int32)
                 % NS)[:, None]
        dp = jnp.concatenate(
            [d2, jnp.broadcast_to(dummy, (nparts, npad))], axis=1
        ).reshape(nparts, nchunk, CH)
        return sp, dp

    src, dst = edge_split(NW, NCHUNK)

    ones16 = jnp.ones((CH, 16), jnp.float32)
    zeros16 = jnp.zeros((RPT, 16), jnp.float32)
    zeros128 = jnp.zeros((RPT, 128), jnp.float32)
    zeros48 = jnp.zeros((RPT, 48), jnp.float32)

    w1p = jnp.pad(W1, ((0, 0), (0, 8)))
    b1p = jnp.pad(b1, (0, 8)).reshape(1, 48)
    wc0p = jnp.pad(Wc0, ((0, 8), (0, 0)))
    wc1p = jnp.pad(Wc1, ((0, 0), (0, 8)))
    b0r = b0.reshape(1, 128)
    bc0r = bc0.reshape(1, 128)
    bc1r = bc1.reshape(1, 40)

    agg128 = _make_agg(128)
    agg48 = _make_agg(48)
    degp = _make_deg()(dst, ones16, zeros16)
    g1 = _t1(x, W0, degp)
    a1 = agg128(g1, src, dst, zeros128)
    g2 = _t2(g1, a1, degp, w1p, b0r)
    a2 = agg48(g2, src, dst, zeros48)
    g3 = _t3(g2, a2, degp, b1p)
    a3 = agg48(g3, src, dst, zeros48)
    g4 = _t4(g3, a3, degp, wc0p, bc0r, wc1p)
    a4 = agg48(g4, src, dst, zeros48)
    out = _t5(g4, a4, degp, bc1r)
    return out


# speedup vs baseline: 1.2910x; 1.2910x over previous
import functools

import jax
import jax.numpy as jnp
from jax import lax
from jax.experimental import pallas as pl
from jax.experimental.pallas import tpu as pltpu
from jax.experimental.pallas import tpu_sc as plsc

N = 10000
E = 320000
NC = 2
NS = 16
NW = NC * NS
EPW = E // NW
CH = 80
NCHUNK = 125
EPT = NCHUNK * CH
NP = N + 16
RPT = 624
REMZ = NP - NS * RPT
REMO = N - NS * RPT

BLK = 1000
GRID = N // BLK

@functools.cache
def _mesh():
    return plsc.VectorSubcoreMesh(core_axis_name="c", subcore_axis_name="s",
                                  num_cores=NC, num_subcores=NS)


@functools.cache
def _make_agg(Dp):
    @functools.partial(
        pl.kernel,
        mesh=_mesh(),
        compiler_params=pltpu.CompilerParams(use_tc_tiling_on_sc=False),
        out_type=jax.ShapeDtypeStruct((NC, N, Dp), jnp.float32),
        scratch_types=[
            pltpu.VMEM((NCHUNK, CH), jnp.int32),
            pltpu.VMEM((NCHUNK, CH), jnp.int32),
            pltpu.VMEM((CH, Dp), jnp.float32),
            pltpu.VMEM((CH, Dp), jnp.float32),
            pltpu.VMEM_SHARED((NP, Dp), jnp.float32),
            pltpu.SemaphoreType.DMA,
            pltpu.SemaphoreType.DMA,
        ],
    )
    def agg(table_hbm, src_hbm, dst_hbm, zeros_hbm, out_hbm,
            src_v, dst_v, b0, b1, acc, g0, g1):
        bufs = [b0, b1]
        gsem = [g0, g1]
        c = lax.axis_index("c")
        s = lax.axis_index("s")
        wid = s * NC + c
        pltpu.sync_copy(zeros_hbm, acc.at[pl.ds(s * RPT, RPT)])

        @pl.when(s == 0)
        def _():
            pltpu.sync_copy(zeros_hbm.at[pl.ds(0, REMZ)],
                            acc.at[pl.ds(NS * RPT, REMZ)])

        pltpu.sync_copy(src_hbm.at[wid], src_v)
        pltpu.sync_copy(dst_hbm.at[wid], dst_v)
        plsc.subcore_barrier()

        def issue_g(i, k):
            pltpu.async_copy(table_hbm.at[src_v.at[i]], bufs[k], gsem[k])

        def wait_g(k):
            pltpu.make_async_copy(table_hbm.at[src_v.at[0]], bufs[k],
                                  gsem[k]).wait()

        issue_g(0, 0)

        def pair(j, _):
            i0 = 2 * j
            issue_g(i0 + 1, 1)
            wait_g(0)
            pltpu.sync_copy(bufs[0], acc.at[dst_v.at[i0]], add=True)

            @pl.when(i0 + 2 < NCHUNK)
            def _():
                issue_g(i0 + 2, 0)

            wait_g(1)
            pltpu.sync_copy(bufs[1], acc.at[dst_v.at[i0 + 1]], add=True)
            return ()

        lax.fori_loop(0, NCHUNK // 2, pair, (), unroll=False)
        if NCHUNK % 2:
            wait_g(0)
            pltpu.sync_copy(bufs[0], acc.at[dst_v.at[NCHUNK - 1]], add=True)
        plsc.subcore_barrier()
        pltpu.sync_copy(acc.at[pl.ds(s * RPT, RPT)],
                        out_hbm.at[c, pl.ds(s * RPT, RPT)])

        @pl.when(s == 0)
        def _():
            pltpu.sync_copy(acc.at[pl.ds(NS * RPT, REMO)],
                            out_hbm.at[c, pl.ds(NS * RPT, REMO)])

    return agg


@functools.cache
def _make_deg():
    @functools.partial(
        pl.kernel,
        mesh=_mesh(),
        compiler_params=pltpu.CompilerParams(use_tc_tiling_on_sc=False),
        out_type=jax.ShapeDtypeStruct((NC, N, 16), jnp.float32),
        scratch_types=[
            pltpu.VMEM((NCHUNK, CH), jnp.int32),
            pltpu.VMEM((CH, 16), jnp.float32),
            pltpu.VMEM_SHARED((NP, 16), jnp.float32),
            pltpu.SemaphoreType.DMA,
        ],
    )
    def deg(dst_hbm, ones_hbm, zeros_hbm, out_hbm, dst_v, ones_v, acc, sem):
        c = lax.axis_index("c")
        s = lax.axis_index("s")
        wid = s * NC + c
        pltpu.sync_copy(zeros_hbm, acc.at[pl.ds(s * RPT, RPT)])

        @pl.when(s == 0)
        def _():
            pltpu.sync_copy(zeros_hbm.at[pl.ds(0, REMZ)],
                            acc.at[pl.ds(NS * RPT, REMZ)])

        pltpu.sync_copy(ones_hbm, ones_v)
        pltpu.sync_copy(dst_hbm.at[wid], dst_v)
        plsc.subcore_barrier()

        def fire(i, _):
            pltpu.async_copy(ones_v, acc.at[dst_v.at[i]], sem, add=True)
            return ()

        lax.fori_loop(0, NCHUNK, fire, (), unroll=False)

        def drain(i, _):
            pltpu.make_async_copy(ones_v, acc.at[dst_v.at[0]], sem).wait()
            return ()

        lax.fori_loop(0, NCHUNK, drain, (), unroll=False)
        plsc.subcore_barrier()
        pltpu.sync_copy(acc.at[pl.ds(s * RPT, RPT)],
                        out_hbm.at[c, pl.ds(s * RPT, RPT)])

        @pl.when(s == 0)
        def _():
            pltpu.sync_copy(acc.at[pl.ds(NS * RPT, REMO)],
                            out_hbm.at[c, pl.ds(NS * RPT, REMO)])

    return deg


def _dinv_of(degp_ref):
    deg = degp_ref[0, :, 0:1] + degp_ref[1, :, 0:1] + 1.0
    return lax.rsqrt(deg)


def _t1_body(x_ref, w0_ref, degp_ref, g1_ref):
    h = jnp.dot(x_ref[...], w0_ref[...], preferred_element_type=jnp.float32)
    g1_ref[...] = _dinv_of(degp_ref) * h


def _t2_body(g1_ref, a1_ref, degp_ref, w1_ref, b0_ref, g2_ref):
    dinv = _dinv_of(degp_ref)
    out1 = jnp.maximum(
        dinv * (a1_ref[0] + a1_ref[1] + g1_ref[...]) + b0_ref[...], 0.0)
    g2_ref[...] = dinv * jnp.dot(out1, w1_ref[...],
                                 preferred_element_type=jnp.float32)


def _t3_body(g2_ref, a2_ref, degp_ref, b1_ref, g3_ref):
    dinv = _dinv_of(degp_ref)
    logits = dinv * (a2_ref[0] + a2_ref[1] + g2_ref[...]) + b1_ref[...]
    pad = lax.broadcasted_iota(jnp.int32, logits.shape, 1) >= 40
    logits = jnp.where(pad, -1e30, logits)
    m = jnp.max(logits, axis=1, keepdims=True)
    e = jnp.exp(logits - m)
    p = e / jnp.sum(e, axis=1, keepdims=True)
    g3_ref[...] = dinv * p


def _t4_body(g3_ref, a3_ref, degp_ref, wc0_ref, bc0_ref, wc1_ref, g4_ref):
    dinv = _dinv_of(degp_ref)
    z = dinv * (a3_ref[0] + a3_ref[1] + g3_ref[...])
    h2 = jnp.maximum(
        jnp.dot(z, wc0_ref[...], preferred_element_type=jnp.float32)
        + bc0_ref[...], 0.0)
    g4_ref[...] = dinv * jnp.dot(h2, wc1_ref[...],
                                 preferred_element_type=jnp.float32)


def _t5_body(g4_ref, a4_ref, degp_ref, bc1_ref, out_ref):
    dinv = _dinv_of(degp_ref)
    out_ref[...] = (dinv * (a4_ref[0] + a4_ref[1] + g4_ref[...]))[:, :40] \
        + bc1_ref[...]


def _row_spec(d):
    return pl.BlockSpec((BLK, d), lambda i: (i, 0))


def _agg_spec(d):
    return pl.BlockSpec((2, BLK, d), lambda i: (0, i, 0))


def _full_spec(shape):
    nd = len(shape)
    return pl.BlockSpec(shape, lambda i: (0,) * nd)


_DEGP = _agg_spec(16)


def _tc_call(body, in_specs, out_d):
    return pl.pallas_call(
        body,
        grid=(GRID,),
        in_specs=in_specs,
        out_specs=_row_spec(out_d),
        out_shape=jax.ShapeDtypeStruct((N, out_d), jnp.float32),
    )


_t1 = _tc_call(_t1_body, [_row_spec(128), _full_spec((128, 128)), _DEGP], 128)
_t2 = _tc_call(_t2_body, [_row_spec(128), _agg_spec(128), _DEGP,
                          _full_spec((128, 48)), _full_spec((1, 128))], 48)
_t3 = _tc_call(_t3_body, [_row_spec(48), _agg_spec(48), _DEGP,
                          _full_spec((1, 48))], 48)
_t4 = _tc_call(_t4_body, [_row_spec(48), _agg_spec(48), _DEGP,
                          _full_spec((48, 128)), _full_spec((1, 128)),
                          _full_spec((128, 48))], 48)
_t5 = _tc_call(_t5_body, [_row_spec(48), _agg_spec(48), _DEGP,
                          _full_spec((1, 40))], 40)


def kernel(x, edge_index, W0, b0, W1, b1, Wc0, bc0, Wc1, bc1):
    def edge_split(nparts, nchunk):
        npad = nchunk * CH - E // nparts
        s2 = edge_index[0].reshape(nparts, E // nparts)
        d2 = edge_index[1].reshape(nparts, E // nparts)
        sp = jnp.pad(s2, ((0, 0), (0, npad))).reshape(nparts, nchunk, CH)
        dummy = (N + jnp.arange(nparts, dtype=jnp.int32)
                 % NS)[:, None]
        dp = jnp.concatenate(
            [d2, jnp.broadcast_to(dummy, (nparts, npad))], axis=1
        ).reshape(nparts, nchunk, CH)
        return sp, dp

    src, dst = edge_split(NW, NCHUNK)

    ones16 = jnp.ones((CH, 16), jnp.float32)
    zeros16 = jnp.zeros((RPT, 16), jnp.float32)
    zeros128 = jnp.zeros((RPT, 128), jnp.float32)
    zeros48 = jnp.zeros((RPT, 48), jnp.float32)

    w1p = jnp.pad(W1, ((0, 0), (0, 8)))
    b1p = jnp.pad(b1, (0, 8)).reshape(1, 48)
    wc0p = jnp.pad(Wc0, ((0, 8), (0, 0)))
    wc1p = jnp.pad(Wc1, ((0, 0), (0, 8)))
    b0r = b0.reshape(1, 128)
    bc0r = bc0.reshape(1, 128)
    bc1r = bc1.reshape(1, 40)

    agg128 = _make_agg(128)
    agg48 = _make_agg(48)
    degp = _make_deg()(dst, ones16, zeros16)
    g1 = _t1(x, W0, degp)
    a1 = agg128(g1, src, dst, zeros128)
    g2 = _t2(g1, a1, degp, w1p, b0r)
    a2 = agg48(g2, src, dst, zeros48)
    g3 = _t3(g2, a2, degp, b1p)
    a3 = agg48(g3, src, dst, zeros48)
    g4 = _t4(g3, a3, degp, wc0p, bc0r, wc1p)
    a4 = agg48(g4, src, dst, zeros48)
    out = _t5(g4, a4, degp, bc1r)
    return out
